# baseline (speedup 1.0000x reference)
import functools

import jax
import jax.numpy as jnp
from jax import lax
from jax.experimental import pallas as pl
from jax.experimental.pallas import tpu as pltpu
from jax.experimental.pallas import tpu_sc as plsc

NU = 100000
NI = 100000
D = 128
H = 64
HH = 32
NC = 2
NS = 16
E = 300000
CH = 128
NCHUNK = 152
BC = 8
EPT = CH * NCHUNK
EPAD = EPT * NS
NPAD = 102400
RROWS = 51200
TRASH = RROWS
ACCR = RROWS + 64
SENT = 100000

_mesh = plsc.VectorSubcoreMesh(
    core_axis_name="c", subcore_axis_name="s", num_cores=NC, num_subcores=NS)
_sc_params = pltpu.CompilerParams(use_tc_tiling_on_sc=False)


def _deg_body(edges, out, acc, idxc, zbuf, obuf):
    c = lax.axis_index("c")
    t = lax.axis_index("s")
    lane = lax.iota(jnp.int32, 16)
    zv = jnp.zeros((16,), jnp.float32)
    e0 = jnp.where(lane == 0, 1.0, 0.0).astype(jnp.float32)

    def fill(i, _):
        zbuf[i, :] = zv
        obuf[i, :] = e0
        return 0
    lax.fori_loop(0, CH, fill, 0)

    for a in range(2):
        arr = 2 * c + a
        def zstep(z, _):
            pltpu.sync_copy(zbuf, acc.at[pl.ds(t * 6400 + z * CH, CH)])
            return 0
        lax.fori_loop(0, 6400 // CH, zstep, 0)
        plsc.subcore_barrier()

        def step(b, _):
            pltpu.sync_copy(edges.at[arr, t, pl.ds(BC * b, BC)], idxc)
            for k in range(BC):
                pltpu.sync_copy(obuf, acc.at[idxc.at[k]], add=True)
            return 0
        lax.fori_loop(0, NCHUNK // BC, step, 0)
        plsc.subcore_barrier()
        def wstep(z, _):
            base = t * 6400 + z * 640
            pltpu.sync_copy(acc.at[pl.ds(base, 640)],
                            out.at[arr, pl.ds(base, 640)])
            return 0
        lax.fori_loop(0, 10, wstep, 0)
        plsc.subcore_barrier()


_deg_call = pl.kernel(
    _deg_body,
    out_type=jax.ShapeDtypeStruct((4, NPAD, 16), jnp.float32),
    mesh=_mesh,
    compiler_params=_sc_params,
    scratch_types=[
        pltpu.VMEM_SHARED((NPAD, 16), jnp.float32),
        pltpu.VMEM((BC, CH), jnp.int32),
        pltpu.VMEM((CH, 16), jnp.float32),
        pltpu.VMEM((CH, 16), jnp.float32),
    ],
)


def _conv_body(h2, edges, out, acc, gix, six, rb0, rb1, rb2,
               s0, s1, s2, se0, se1, *, er):
    c = lax.axis_index("c")
    t = lax.axis_index("s")
    zv = jnp.zeros((16,), jnp.float32)

    for r in range(2):
        lo = r * RROWS

        def fillz(i, _):
            rb0[i, pl.ds(0, 16)] = zv
            rb0[i, pl.ds(16, 16)] = zv
            return 0
        lax.fori_loop(0, CH, fillz, 0)

        def zstep(z, _):
            pltpu.sync_copy(rb0, acc.at[pl.ds(t * 3204 + z * CH, CH)])
            return 0
        lax.fori_loop(0, 3200 // CH, zstep, 0)
        pltpu.sync_copy(rb0.at[pl.ds(0, 4)],
                        acc.at[pl.ds(t * 3204 + 3200, 4)])
        plsc.subcore_barrier()

        rbs = (rb0, rb1, rb2)
        sems = (s0, s1, s2)

        def blk(b, _):
            e0 = pltpu.async_copy(
                edges.at[er, t, pl.ds(BC * b, BC)], gix, se0)
            e1 = pltpu.async_copy(
                edges.at[er + 1, t, pl.ds(BC * b, BC)], six, se1)
            e0.wait()
            e1.wait()
            for rr in range(BC):
                for l in range(CH // 16):
                    sl = pl.ds(16 * l, 16)
                    s = gix[rr, sl]
                    gix[rr, sl] = 2 * jnp.minimum(s, NU - 1) + c
                    d = six[rr, sl]
                    ok = (d >= lo) & (d < lo + RROWS)
                    six[rr, sl] = jnp.where(ok, d - lo, TRASH)
            pend = []
            for k in range(3):
                pend.append(
                    pltpu.async_copy(h2.at[gix.at[k]], rbs[k], sems[k]))
            for k in range(BC):
                pend[k].wait()
                pltpu.sync_copy(rbs[k % 3], acc.at[six.at[k]], add=True)
                if k + 3 < BC:
                    pend.append(pltpu.async_copy(
                        h2.at[gix.at[k + 3]], rbs[(k + 3) % 3],
                        sems[(k + 3) % 3]))
            return 0
        lax.fori_loop(0, NCHUNK // BC, blk, 0)
        plsc.subcore_barrier()

        def wstep(k, _):
            off = t * 3200 + k * 640
            pltpu.sync_copy(acc.at[pl.ds(off, 640)],
                            out.at[c, pl.ds(lo + off, 640)])
            return 0
        lax.fori_loop(0, 5, wstep, 0)
        plsc.subcore_barrier()


def _make_conv(er):
    return pl.kernel(
        functools.partial(_conv_body, er=er),
        out_type=jax.ShapeDtypeStruct((NC, NPAD, HH), jnp.float32),
        mesh=_mesh,
        compiler_params=_sc_params,
        scratch_types=[
            pltpu.VMEM_SHARED((ACCR, HH), jnp.float32),
            pltpu.VMEM((BC, CH), jnp.int32),
            pltpu.VMEM((BC, CH), jnp.int32),
            pltpu.VMEM((CH, HH), jnp.float32),
            pltpu.VMEM((CH, HH), jnp.float32),
            pltpu.VMEM((CH, HH), jnp.float32),
            pltpu.SemaphoreType.DMA,
            pltpu.SemaphoreType.DMA,
            pltpu.SemaphoreType.DMA,
            pltpu.SemaphoreType.DMA,
            pltpu.SemaphoreType.DMA,
        ],
    )


_conv_ui = _make_conv(0)
_conv_iu = _make_conv(2)

_BR = 1000
_NBLK = NU // _BR


def _mm1_body(x_ref, w_ref, dg_ref, o_ref):
    a = lax.rsqrt(jnp.maximum(dg_ref[...], 1.0))
    o_ref[...] = jnp.dot(x_ref[...], w_ref[...],
                         preferred_element_type=jnp.float32) * a


_mm1 = pl.pallas_call(
    _mm1_body,
    grid=(_NBLK,),
    in_specs=[
        pl.BlockSpec((_BR, D), lambda i: (i, 0)),
        pl.BlockSpec((D, H), lambda i: (0, 0)),
        pl.BlockSpec((_BR, 1), lambda i: (i, 0)),
    ],
    out_specs=pl.BlockSpec((_BR, H), lambda i: (i, 0)),
    out_shape=jax.ShapeDtypeStruct((NU, H), jnp.float32),
)


def _post_body(acc_ref, dd_ref, b1_ref, w2_ref, dn_ref, o_ref):
    h = jnp.concatenate([acc_ref[0], acc_ref[1]], axis=-1)
    bd = lax.rsqrt(jnp.maximum(dd_ref[...], 1.0))
    h = jnp.maximum(h * bd + b1_ref[...], 0.0)
    an = lax.rsqrt(jnp.maximum(dn_ref[...], 1.0))
    o_ref[...] = jnp.dot(h, w2_ref[...],
                         preferred_element_type=jnp.float32) * an


_post = pl.pallas_call(
    _post_body,
    grid=(_NBLK,),
    in_specs=[
        pl.BlockSpec((NC, _BR, HH), lambda i: (0, i, 0)),
        pl.BlockSpec((_BR, 1), lambda i: (i, 0)),
        pl.BlockSpec((1, H), lambda i: (0, 0)),
        pl.BlockSpec((H, H), lambda i: (0, 0)),
        pl.BlockSpec((_BR, 1), lambda i: (i, 0)),
    ],
    out_specs=pl.BlockSpec((_BR, H), lambda i: (i, 0)),
    out_shape=jax.ShapeDtypeStruct((NU, H), jnp.float32),
)


def _head_body(acc_ref, dd_ref, b2_ref, wl_ref, bl_ref, o_ref):
    h = jnp.concatenate([acc_ref[0], acc_ref[1]], axis=-1)
    bd = lax.rsqrt(jnp.maximum(dd_ref[...], 1.0))
    h = jnp.maximum(h * bd + b2_ref[...], 0.0)
    o_ref[...] = jnp.dot(h, wl_ref[...],
                         preferred_element_type=jnp.float32) + bl_ref[...]


_head = pl.pallas_call(
    _head_body,
    grid=(_NBLK,),
    in_specs=[
        pl.BlockSpec((NC, _BR, HH), lambda i: (0, i, 0)),
        pl.BlockSpec((_BR, 1), lambda i: (i, 0)),
        pl.BlockSpec((1, H), lambda i: (0, 0)),
        pl.BlockSpec((H, 2), lambda i: (0, 0)),
        pl.BlockSpec((1, 2), lambda i: (0, 0)),
    ],
    out_specs=pl.BlockSpec((_BR, 2), lambda i: (i, 0)),
    out_shape=jax.ShapeDtypeStruct((NU, 2), jnp.float32),
)


def kernel(x_user, x_item, edge_index_ui, edge_index_iu,
           W1_ui, b1_ui, W1_iu, b1_iu,
           W2_ui, b2_ui, W2_iu, b2_iu,
           Wl_user, bl_user, Wl_item, bl_item):
    pad = jnp.full((EPAD - E,), SENT, jnp.int32)

    def prep(v):
        return jnp.concatenate([v.astype(jnp.int32), pad])

    edges = jnp.stack([
        prep(edge_index_ui[0]), prep(edge_index_ui[1]),
        prep(edge_index_iu[0]), prep(edge_index_iu[1]),
    ]).reshape(4, NS, NCHUNK, CH)

    deg = _deg_call(edges)
    dgu_src = deg[0, :NU, 0:1]
    dgi_dst = deg[1, :NI, 0:1]
    dgi_src = deg[2, :NI, 0:1]
    dgu_dst = deg[3, :NU, 0:1]

    b1_ui2 = b1_ui[None, :]
    b1_iu2 = b1_iu[None, :]
    b2_ui2 = b2_ui[None, :]
    b2_iu2 = b2_iu[None, :]
    bl_u2 = bl_user[None, :]
    bl_i2 = bl_item[None, :]

    h1u = _mm1(x_user, W1_ui, dgu_src)
    acc1i = _conv_ui(h1u.reshape(2 * NU, HH), edges)
    h1i = _mm1(x_item, W1_iu, dgi_src)
    acc1u = _conv_iu(h1i.reshape(2 * NI, HH), edges)

    hA = _post(acc1i, dgi_dst, b1_ui2, W2_iu, dgi_src)
    acc2u = _conv_iu(hA.reshape(2 * NI, HH), edges)
    out_user = _head(acc2u, dgu_dst, b2_iu2, Wl_user, bl_u2)

    hB = _post(acc1u, dgu_dst, b1_iu2, W2_ui, dgu_src)
    acc2i = _conv_ui(hB.reshape(2 * NU, HH), edges)
    out_item = _head(acc2i, dgi_dst, b2_ui2, Wl_item, bl_i2)

    return (out_user, out_item)


# speedup vs baseline: 1.3950x; 1.3950x over previous
import functools

import jax
import jax.numpy as jnp
from jax import lax
from jax.experimental import pallas as pl
from jax.experimental.pallas import tpu as pltpu
from jax.experimental.pallas import tpu_sc as plsc

NU = 100000
NI = 100000
D = 128
H = 64
HH = 32
NC = 2
NS = 16
E = 300000
CH = 128
NCHUNK = 148
BC = 4
EPT = CH * NCHUNK
EPAD = EPT * NS
NPAD = 102400
RROWS = 51200
TRASH = RROWS
ACCR = RROWS + 64
SENT = 100000

_mesh = plsc.VectorSubcoreMesh(
    core_axis_name="c", subcore_axis_name="s", num_cores=NC, num_subcores=NS)
_sc_params = pltpu.CompilerParams(use_tc_tiling_on_sc=False)


def _deg_body(edges, out, acc, idxc, zbuf, obuf):
    c = lax.axis_index("c")
    t = lax.axis_index("s")
    lane = lax.iota(jnp.int32, 16)
    zv = jnp.zeros((16,), jnp.float32)
    e0 = jnp.where(lane == 0, 1.0, 0.0).astype(jnp.float32)

    def fill(i, _):
        zbuf[i, :] = zv
        obuf[i, :] = e0
        return 0
    lax.fori_loop(0, CH, fill, 0)

    for a in range(2):
        arr = 2 * c + a
        def zstep(z, _):
            pltpu.sync_copy(zbuf, acc.at[pl.ds(t * 6400 + z * CH, CH)])
            return 0
        lax.fori_loop(0, 6400 // CH, zstep, 0)
        plsc.subcore_barrier()

        def step(b, _):
            pltpu.sync_copy(edges.at[arr, t, pl.ds(BC * b, BC)], idxc)
            for k in range(BC):
                pltpu.sync_copy(obuf, acc.at[idxc.at[k]], add=True)
            return 0
        lax.fori_loop(0, NCHUNK // BC, step, 0)
        plsc.subcore_barrier()
        def wstep(z, _):
            base = t * 6400 + z * 640
            pltpu.sync_copy(acc.at[pl.ds(base, 640)],
                            out.at[arr, pl.ds(base, 640)])
            return 0
        lax.fori_loop(0, 10, wstep, 0)
        plsc.subcore_barrier()


_deg_call = pl.kernel(
    _deg_body,
    out_type=jax.ShapeDtypeStruct((4, NPAD, 16), jnp.float32),
    mesh=_mesh,
    compiler_params=_sc_params,
    scratch_types=[
        pltpu.VMEM_SHARED((NPAD, 16), jnp.float32),
        pltpu.VMEM((BC, CH), jnp.int32),
        pltpu.VMEM((CH, 16), jnp.float32),
        pltpu.VMEM((CH, 16), jnp.float32),
    ],
)


def _conv_body(h2, edges, out, acc, gix, six, rb0, rb1, rb2, rb3,
               s0, s1, s2, s3, *, er):
    c = lax.axis_index("c")
    t = lax.axis_index("s")
    zv = jnp.zeros((16,), jnp.float32)

    for r in range(2):
        lo = r * RROWS

        def fillz(i, _):
            rb0[i, pl.ds(0, 16)] = zv
            rb0[i, pl.ds(16, 16)] = zv
            return 0
        lax.fori_loop(0, CH, fillz, 0)

        def zstep(z, _):
            pltpu.sync_copy(rb0, acc.at[pl.ds(t * 3204 + z * CH, CH)])
            return 0
        lax.fori_loop(0, 3200 // CH, zstep, 0)
        pltpu.sync_copy(rb0.at[pl.ds(0, 4)],
                        acc.at[pl.ds(t * 3204 + 3200, 4)])
        plsc.subcore_barrier()

        rbs = (rb0, rb1, rb2, rb3)
        sems = (s0, s1, s2, s3)

        def blk(b, _):
            pltpu.sync_copy(edges.at[er, t, pl.ds(BC * b, BC)], gix)
            pltpu.sync_copy(edges.at[er + 1, t, pl.ds(BC * b, BC)], six)
            for rr in range(BC):
                for l in range(CH // 16):
                    sl = pl.ds(16 * l, 16)
                    s = gix[rr, sl]
                    gix[rr, sl] = 2 * jnp.minimum(s, NU - 1) + c
                    d = six[rr, sl]
                    ok = (d >= lo) & (d < lo + RROWS)
                    six[rr, sl] = jnp.where(ok, d - lo, TRASH)
            pend = []
            for k in range(BC):
                pend.append(
                    pltpu.async_copy(h2.at[gix.at[k]], rbs[k], sems[k]))
            for k in range(BC):
                pend[k].wait()
                pltpu.sync_copy(rbs[k], acc.at[six.at[k]], add=True)
            return 0
        lax.fori_loop(0, NCHUNK // BC, blk, 0)
        plsc.subcore_barrier()

        def wstep(k, _):
            off = t * 3200 + k * 640
            pltpu.sync_copy(acc.at[pl.ds(off, 640)],
                            out.at[c, pl.ds(lo + off, 640)])
            return 0
        lax.fori_loop(0, 5, wstep, 0)
        plsc.subcore_barrier()


def _make_conv(er):
    return pl.kernel(
        functools.partial(_conv_body, er=er),
        out_type=jax.ShapeDtypeStruct((NC, NPAD, HH), jnp.float32),
        mesh=_mesh,
        compiler_params=_sc_params,
        scratch_types=[
            pltpu.VMEM_SHARED((ACCR, HH), jnp.float32),
            pltpu.VMEM((BC, CH), jnp.int32),
            pltpu.VMEM((BC, CH), jnp.int32),
            pltpu.VMEM((CH, HH), jnp.float32),
            pltpu.VMEM((CH, HH), jnp.float32),
            pltpu.VMEM((CH, HH), jnp.float32),
            pltpu.VMEM((CH, HH), jnp.float32),
            pltpu.SemaphoreType.DMA,
            pltpu.SemaphoreType.DMA,
            pltpu.SemaphoreType.DMA,
            pltpu.SemaphoreType.DMA,
        ],
    )


_conv_ui = _make_conv(0)
_conv_iu = _make_conv(2)

_BR = 1000
_NBLK = NU // _BR


def _mm1_body(x_ref, w_ref, dg_ref, o_ref):
    a = lax.rsqrt(jnp.maximum(dg_ref[...], 1.0))
    o_ref[...] = jnp.dot(x_ref[...], w_ref[...],
                         preferred_element_type=jnp.float32) * a


_mm1 = pl.pallas_call(
    _mm1_body,
    grid=(_NBLK,),
    in_specs=[
        pl.BlockSpec((_BR, D), lambda i: (i, 0)),
        pl.BlockSpec((D, H), lambda i: (0, 0)),
        pl.BlockSpec((_BR, 1), lambda i: (i, 0)),
    ],
    out_specs=pl.BlockSpec((_BR, H), lambda i: (i, 0)),
    out_shape=jax.ShapeDtypeStruct((NU, H), jnp.float32),
)


def _post_body(acc_ref, dd_ref, b1_ref, w2_ref, dn_ref, o_ref):
    h = jnp.concatenate([acc_ref[0], acc_ref[1]], axis=-1)
    bd = lax.rsqrt(jnp.maximum(dd_ref[...], 1.0))
    h = jnp.maximum(h * bd + b1_ref[...], 0.0)
    an = lax.rsqrt(jnp.maximum(dn_ref[...], 1.0))
    o_ref[...] = jnp.dot(h, w2_ref[...],
                         preferred_element_type=jnp.float32) * an


_post = pl.pallas_call(
    _post_body,
    grid=(_NBLK,),
    in_specs=[
        pl.BlockSpec((NC, _BR, HH), lambda i: (0, i, 0)),
        pl.BlockSpec((_BR, 1), lambda i: (i, 0)),
        pl.BlockSpec((1, H), lambda i: (0, 0)),
        pl.BlockSpec((H, H), lambda i: (0, 0)),
        pl.BlockSpec((_BR, 1), lambda i: (i, 0)),
    ],
    out_specs=pl.BlockSpec((_BR, H), lambda i: (i, 0)),
    out_shape=jax.ShapeDtypeStruct((NU, H), jnp.float32),
)


def _head_body(acc_ref, dd_ref, b2_ref, wl_ref, bl_ref, o_ref):
    h = jnp.concatenate([acc_ref[0], acc_ref[1]], axis=-1)
    bd = lax.rsqrt(jnp.maximum(dd_ref[...], 1.0))
    h = jnp.maximum(h * bd + b2_ref[...], 0.0)
    o_ref[...] = jnp.dot(h, wl_ref[...],
                         preferred_element_type=jnp.float32) + bl_ref[...]


_head = pl.pallas_call(
    _head_body,
    grid=(_NBLK,),
    in_specs=[
        pl.BlockSpec((NC, _BR, HH), lambda i: (0, i, 0)),
        pl.BlockSpec((_BR, 1), lambda i: (i, 0)),
        pl.BlockSpec((1, H), lambda i: (0, 0)),
        pl.BlockSpec((H, 2), lambda i: (0, 0)),
        pl.BlockSpec((1, 2), lambda i: (0, 0)),
    ],
    out_specs=pl.BlockSpec((_BR, 2), lambda i: (i, 0)),
    out_shape=jax.ShapeDtypeStruct((NU, 2), jnp.float32),
)


def kernel(x_user, x_item, edge_index_ui, edge_index_iu,
           W1_ui, b1_ui, W1_iu, b1_iu,
           W2_ui, b2_ui, W2_iu, b2_iu,
           Wl_user, bl_user, Wl_item, bl_item):
    pad = jnp.full((EPAD - E,), SENT, jnp.int32)

    def prep(v):
        return jnp.concatenate([v.astype(jnp.int32), pad])

    edges = jnp.stack([
        prep(edge_index_ui[0]), prep(edge_index_ui[1]),
        prep(edge_index_iu[0]), prep(edge_index_iu[1]),
    ]).reshape(4, NS, NCHUNK, CH)

    deg = _deg_call(edges)
    dgu_src = deg[0, :NU, 0:1]
    dgi_dst = deg[1, :NI, 0:1]
    dgi_src = deg[2, :NI, 0:1]
    dgu_dst = deg[3, :NU, 0:1]

    b1_ui2 = b1_ui[None, :]
    b1_iu2 = b1_iu[None, :]
    b2_ui2 = b2_ui[None, :]
    b2_iu2 = b2_iu[None, :]
    bl_u2 = bl_user[None, :]
    bl_i2 = bl_item[None, :]

    h1u = _mm1(x_user, W1_ui, dgu_src)
    acc1i = _conv_ui(h1u.reshape(2 * NU, HH), edges)
    h1i = _mm1(x_item, W1_iu, dgi_src)
    acc1u = _conv_iu(h1i.reshape(2 * NI, HH), edges)

    hA = _post(acc1i, dgi_dst, b1_ui2, W2_iu, dgi_src)
    acc2u = _conv_iu(hA.reshape(2 * NI, HH), edges)
    out_user = _head(acc2u, dgu_dst, b2_iu2, Wl_user, bl_u2)

    hB = _post(acc1u, dgu_dst, b1_iu2, W2_ui, dgu_src)
    acc2i = _conv_ui(hB.reshape(2 * NU, HH), edges)
    out_item = _head(acc2i, dgi_dst, b2_ui2, Wl_item, bl_i2)

    return (out_user, out_item)
